# baseline (speedup 1.0000x reference)
import jax
import jax.numpy as jnp
from jax.experimental import pallas as pl
from jax.experimental.pallas import tpu as pltpu


_SAMPLES_PER_BLOCK = 2
_NBUF = 8


def _add_noise_manual(t_ref, sa_ref, sb_ref, x_hbm, n_hbm, o_hbm,
                      xb, nb, ob, sx, sn, so):
    i = pl.program_id(0)
    steps = pl.num_programs(0)
    br = xb.shape[1]

    def start_inputs(j, slot):
        off = j * br
        pltpu.make_async_copy(
            x_hbm.at[pl.ds(off, br), :], xb.at[slot], sx.at[slot]).start()
        pltpu.make_async_copy(
            n_hbm.at[pl.ds(off, br), :], nb.at[slot], sn.at[slot]).start()

    slot = jax.lax.rem(i, _NBUF)

    @pl.when(i == 0)
    def _():
        for j in range(_NBUF):
            start_inputs(j, j)

    @pl.when(i >= _NBUF)
    def _():
        pltpu.make_async_copy(
            ob.at[slot], o_hbm.at[pl.ds(0, br), :], so.at[slot]).wait()

    pltpu.make_async_copy(
        x_hbm.at[pl.ds(0, br), :], xb.at[slot], sx.at[slot]).wait()
    pltpu.make_async_copy(
        n_hbm.at[pl.ds(0, br), :], nb.at[slot], sn.at[slot]).wait()

    xv = xb.at[slot]
    nv = nb.at[slot]
    ov = ob.at[slot]
    rows = br // _SAMPLES_PER_BLOCK
    for k in range(_SAMPLES_PER_BLOCK):
        tt = t_ref[i * _SAMPLES_PER_BLOCK + k]
        a = sa_ref[tt]
        b = sb_ref[tt]
        sl = pl.ds(k * rows, rows)
        ov[sl, :] = a * xv[sl, :] + b * nv[sl, :]

    pltpu.make_async_copy(
        ob.at[slot], o_hbm.at[pl.ds(i * br, br), :], so.at[slot]).start()

    @pl.when(i + _NBUF < steps)
    def _():
        start_inputs(i + _NBUF, slot)

    @pl.when(i == steps - 1)
    def _():
        for s in range(_NBUF):
            pltpu.make_async_copy(
                ob.at[s], o_hbm.at[pl.ds(0, br), :], so.at[s]).wait()


def kernel(x0, noise, t, sqrt_alphas_cumprod, sqrt_one_minus_alphas_cumprod):
    n, c, h, w = x0.shape
    rows = c * h
    x2 = x0.reshape(n * rows, w)
    n2 = noise.reshape(n * rows, w)
    blk_rows = rows * _SAMPLES_PER_BLOCK
    steps = n // _SAMPLES_PER_BLOCK

    out = pl.pallas_call(
        _add_noise_manual,
        grid_spec=pltpu.PrefetchScalarGridSpec(
            num_scalar_prefetch=3,
            grid=(steps,),
            in_specs=[
                pl.BlockSpec(memory_space=pl.ANY),
                pl.BlockSpec(memory_space=pl.ANY),
            ],
            out_specs=pl.BlockSpec(memory_space=pl.ANY),
            scratch_shapes=[
                pltpu.VMEM((_NBUF, blk_rows, w), jnp.float32),
                pltpu.VMEM((_NBUF, blk_rows, w), jnp.float32),
                pltpu.VMEM((_NBUF, blk_rows, w), jnp.float32),
                pltpu.SemaphoreType.DMA((_NBUF,)),
                pltpu.SemaphoreType.DMA((_NBUF,)),
                pltpu.SemaphoreType.DMA((_NBUF,)),
            ],
        ),
        out_shape=jax.ShapeDtypeStruct((n * rows, w), x0.dtype),
        compiler_params=pltpu.CompilerParams(
            dimension_semantics=("arbitrary",),
        ),
    )(t, sqrt_alphas_cumprod, sqrt_one_minus_alphas_cumprod, x2, n2)
    return out.reshape(n, c, h, w)


# speedup vs baseline: 1.0002x; 1.0002x over previous
import jax
import jax.numpy as jnp
from jax.experimental import pallas as pl
from jax.experimental.pallas import tpu as pltpu


_SAMPLES_PER_BLOCK = 1
_NBUF = 12


def _add_noise_manual(t_ref, sa_ref, sb_ref, x_hbm, n_hbm, o_hbm,
                      xb, nb, ob, sx, sn, so):
    i = pl.program_id(0)
    steps = pl.num_programs(0)
    br = xb.shape[1]

    def start_inputs(j, slot):
        off = j * br
        pltpu.make_async_copy(
            x_hbm.at[pl.ds(off, br), :], xb.at[slot], sx.at[slot]).start()
        pltpu.make_async_copy(
            n_hbm.at[pl.ds(off, br), :], nb.at[slot], sn.at[slot]).start()

    slot = jax.lax.rem(i, _NBUF)

    @pl.when(i == 0)
    def _():
        for j in range(_NBUF):
            start_inputs(j, j)

    @pl.when(i >= _NBUF)
    def _():
        pltpu.make_async_copy(
            ob.at[slot], o_hbm.at[pl.ds(0, br), :], so.at[slot]).wait()

    pltpu.make_async_copy(
        x_hbm.at[pl.ds(0, br), :], xb.at[slot], sx.at[slot]).wait()
    pltpu.make_async_copy(
        n_hbm.at[pl.ds(0, br), :], nb.at[slot], sn.at[slot]).wait()

    xv = xb.at[slot]
    nv = nb.at[slot]
    ov = ob.at[slot]
    rows = br // _SAMPLES_PER_BLOCK
    for k in range(_SAMPLES_PER_BLOCK):
        tt = t_ref[i * _SAMPLES_PER_BLOCK + k]
        a = sa_ref[tt]
        b = sb_ref[tt]
        sl = pl.ds(k * rows, rows)
        ov[sl, :] = a * xv[sl, :] + b * nv[sl, :]

    pltpu.make_async_copy(
        ob.at[slot], o_hbm.at[pl.ds(i * br, br), :], so.at[slot]).start()

    @pl.when(i + _NBUF < steps)
    def _():
        start_inputs(i + _NBUF, slot)

    @pl.when(i == steps - 1)
    def _():
        for s in range(_NBUF):
            pltpu.make_async_copy(
                ob.at[s], o_hbm.at[pl.ds(0, br), :], so.at[s]).wait()


def kernel(x0, noise, t, sqrt_alphas_cumprod, sqrt_one_minus_alphas_cumprod):
    n, c, h, w = x0.shape
    rows = c * h
    x2 = x0.reshape(n * rows, w)
    n2 = noise.reshape(n * rows, w)
    blk_rows = rows * _SAMPLES_PER_BLOCK
    steps = n // _SAMPLES_PER_BLOCK

    out = pl.pallas_call(
        _add_noise_manual,
        grid_spec=pltpu.PrefetchScalarGridSpec(
            num_scalar_prefetch=3,
            grid=(steps,),
            in_specs=[
                pl.BlockSpec(memory_space=pl.ANY),
                pl.BlockSpec(memory_space=pl.ANY),
            ],
            out_specs=pl.BlockSpec(memory_space=pl.ANY),
            scratch_shapes=[
                pltpu.VMEM((_NBUF, blk_rows, w), jnp.float32),
                pltpu.VMEM((_NBUF, blk_rows, w), jnp.float32),
                pltpu.VMEM((_NBUF, blk_rows, w), jnp.float32),
                pltpu.SemaphoreType.DMA((_NBUF,)),
                pltpu.SemaphoreType.DMA((_NBUF,)),
                pltpu.SemaphoreType.DMA((_NBUF,)),
            ],
        ),
        out_shape=jax.ShapeDtypeStruct((n * rows, w), x0.dtype),
        compiler_params=pltpu.CompilerParams(
            dimension_semantics=("arbitrary",),
        ),
    )(t, sqrt_alphas_cumprod, sqrt_one_minus_alphas_cumprod, x2, n2)
    return out.reshape(n, c, h, w)


# speedup vs baseline: 1.0028x; 1.0026x over previous
import jax
import jax.numpy as jnp
from jax.experimental import pallas as pl
from jax.experimental.pallas import tpu as pltpu


_SAMPLES_PER_BLOCK = 2
_NBUF = 6


def _add_noise_manual(t_ref, sa_ref, sb_ref, x_hbm, n_hbm, o_hbm,
                      xb, nb, ob, sx, sn, so):
    i = pl.program_id(0)
    steps = pl.num_programs(0)
    br = xb.shape[1]

    def start_inputs(j, slot):
        off = j * br
        pltpu.make_async_copy(
            x_hbm.at[pl.ds(off, br), :], xb.at[slot], sx.at[slot]).start()
        pltpu.make_async_copy(
            n_hbm.at[pl.ds(off, br), :], nb.at[slot], sn.at[slot]).start()

    slot = jax.lax.rem(i, _NBUF)

    @pl.when(i == 0)
    def _():
        for j in range(_NBUF):
            start_inputs(j, j)

    @pl.when(i >= _NBUF)
    def _():
        pltpu.make_async_copy(
            ob.at[slot], o_hbm.at[pl.ds(0, br), :], so.at[slot]).wait()

    pltpu.make_async_copy(
        x_hbm.at[pl.ds(0, br), :], xb.at[slot], sx.at[slot]).wait()
    pltpu.make_async_copy(
        n_hbm.at[pl.ds(0, br), :], nb.at[slot], sn.at[slot]).wait()

    xv = xb.at[slot]
    nv = nb.at[slot]
    ov = ob.at[slot]
    rows = br // _SAMPLES_PER_BLOCK
    for k in range(_SAMPLES_PER_BLOCK):
        tt = t_ref[i * _SAMPLES_PER_BLOCK + k]
        a = sa_ref[tt]
        b = sb_ref[tt]
        sl = pl.ds(k * rows, rows)
        ov[sl, :] = a * xv[sl, :] + b * nv[sl, :]

    pltpu.make_async_copy(
        ob.at[slot], o_hbm.at[pl.ds(i * br, br), :], so.at[slot]).start()

    @pl.when(i + _NBUF < steps)
    def _():
        start_inputs(i + _NBUF, slot)

    @pl.when(i == steps - 1)
    def _():
        for s in range(_NBUF):
            pltpu.make_async_copy(
                ob.at[s], o_hbm.at[pl.ds(0, br), :], so.at[s]).wait()


def kernel(x0, noise, t, sqrt_alphas_cumprod, sqrt_one_minus_alphas_cumprod):
    n, c, h, w = x0.shape
    rows = c * h
    x2 = x0.reshape(n * rows, w)
    n2 = noise.reshape(n * rows, w)
    blk_rows = rows * _SAMPLES_PER_BLOCK
    steps = n // _SAMPLES_PER_BLOCK

    out = pl.pallas_call(
        _add_noise_manual,
        grid_spec=pltpu.PrefetchScalarGridSpec(
            num_scalar_prefetch=3,
            grid=(steps,),
            in_specs=[
                pl.BlockSpec(memory_space=pl.ANY),
                pl.BlockSpec(memory_space=pl.ANY),
            ],
            out_specs=pl.BlockSpec(memory_space=pl.ANY),
            scratch_shapes=[
                pltpu.VMEM((_NBUF, blk_rows, w), jnp.float32),
                pltpu.VMEM((_NBUF, blk_rows, w), jnp.float32),
                pltpu.VMEM((_NBUF, blk_rows, w), jnp.float32),
                pltpu.SemaphoreType.DMA((_NBUF,)),
                pltpu.SemaphoreType.DMA((_NBUF,)),
                pltpu.SemaphoreType.DMA((_NBUF,)),
            ],
        ),
        out_shape=jax.ShapeDtypeStruct((n * rows, w), x0.dtype),
        compiler_params=pltpu.CompilerParams(
            dimension_semantics=("arbitrary",),
        ),
    )(t, sqrt_alphas_cumprod, sqrt_one_minus_alphas_cumprod, x2, n2)
    return out.reshape(n, c, h, w)
